# baseline (speedup 1.0000x reference)
import functools

import jax
import jax.numpy as jnp
from jax.experimental import pallas as pl
from jax.experimental.pallas import tpu as pltpu

_TOPK = 3
_BIGF = 3.0e38
_TQ = 256
_TK = 2048


def _lexmin_tree(v, i):
    while v.shape[0] >= 16:
        q = v.shape[0] // 4
        v0, v1, v2, v3 = v[:q], v[q:2 * q], v[2 * q:3 * q], v[3 * q:]
        j0, j1, j2, j3 = i[:q], i[q:2 * q], i[2 * q:3 * q], i[3 * q:]
        ca = v1 < v0
        va = jnp.where(ca, v1, v0)
        ja = jnp.where(ca, j1, j0)
        cb = v3 < v2
        vb = jnp.where(cb, v3, v2)
        jb = jnp.where(cb, j3, j2)
        cc = vb < va
        v = jnp.where(cc, vb, va)
        i = jnp.where(cc, jb, ja)
    while v.shape[0] > 1:
        h = v.shape[0] // 2
        c = v[h:] < v[:h]
        i = jnp.where(c, i[h:], i[:h])
        v = jnp.where(c, v[h:], v[:h])
    return v, i


def _tile_top3(d, ic):
    n = d.shape[0]
    q = n // 4
    ia = ic[:q]
    vals, idxs = [], []
    for j in range(_TOPK):
        v0, v1, v2, v3 = d[:q], d[q:2 * q], d[2 * q:3 * q], d[3 * q:]
        ca = v1 < v0
        va = jnp.where(ca, v1, v0)
        ja = jnp.where(ca, ia + q, ia)
        cb = v3 < v2
        vb = jnp.where(cb, v3, v2)
        jb = jnp.where(cb, ia + 3 * q, ia + 2 * q)
        cc = vb < va
        v, i = _lexmin_tree(jnp.where(cc, vb, va), jnp.where(cc, jb, ja))
        vals.append(v)
        idxs.append(i)
        if j < _TOPK - 1:
            d = jnp.where(ic == i, _BIGF, d)
    return jnp.concatenate(vals, axis=0), jnp.concatenate(idxs, axis=0)


def _merge_top3(cv, ci):
    vals, idxs = [], []
    pos = jax.lax.broadcasted_iota(jnp.int32, (8, 1), 0)
    for j in range(_TOPK):
        v, i = _lexmin_tree(cv, ci)
        _, p = _lexmin_tree(cv, jnp.broadcast_to(pos, cv.shape))
        vals.append(v)
        idxs.append(i)
        if j < _TOPK - 1:
            cv = jnp.where(pos == p, _BIGF, cv)
    return jnp.concatenate(vals, axis=0), jnp.concatenate(idxs, axis=0)


def _retr_kernel(nk, img_ref, keys_ref, wt_ref, b_ref, ksq_ref,
                 outv_ref, outi_ref, proj_ref, qsq_ref, rv_ref, ri_ref,
                 mm_ref):
    ki = pl.program_id(1)

    @pl.when(ki == 0)
    def _project():
        p = jax.lax.dot_general(
            img_ref[...], wt_ref[...], (((1,), (0,)), ((), ())),
            preferred_element_type=jnp.float32) + b_ref[...]
        proj_ref[...] = p.astype(jnp.bfloat16)
        qsq_ref[...] = jax.lax.dot_general(
            jnp.ones((1, p.shape[1]), jnp.float32), p * p,
            (((1,), (1,)), ((), ())),
            preferred_element_type=jnp.float32,
            precision=jax.lax.Precision.HIGHEST)

    kj = jnp.maximum(ki - 1, 0)
    mm2 = mm_ref[(ki + 1) % 2]
    d = (qsq_ref[...] + ksq_ref[...]) - mm2

    ic = jax.lax.broadcasted_iota(jnp.int32, (_TK, 1), 0)
    tv, ti = _tile_top3(d, ic)
    ti = kj * _TK + ti

    padv = jnp.full((2, _TQ), _BIGF, jnp.float32)
    padi = jnp.zeros((2, _TQ), jnp.int32)
    cv = jnp.concatenate([rv_ref[...], tv, padv], axis=0)
    ci = jnp.concatenate([ri_ref[...], ti, padi], axis=0)
    nv, ni = _merge_top3(cv, ci)
    rv_ref[...] = nv
    ri_ref[...] = ni

    mm_ref[ki % 2] = jax.lax.dot_general(
        keys_ref[...], proj_ref[...], (((1,), (1,)), ((), ())),
        preferred_element_type=jnp.float32)

    @pl.when(ki == 0)
    def _seed():
        rv_ref[...] = jnp.full((_TOPK, _TQ), _BIGF, jnp.float32)
        ri_ref[...] = jnp.zeros((_TOPK, _TQ), jnp.int32)

    outv_ref[0] = -rv_ref[...]
    outi_ref[0] = ri_ref[...]


def kernel(image_emb, keys, W, b):
    Q, Din = image_emb.shape
    K, D = keys.shape
    nq = Q // _TQ
    nk = (K + _TK - 1) // _TK
    wt = W.T
    b2 = b.reshape(1, D)
    ksq = jnp.sum(keys * keys, axis=1)[:, None]
    pad = nk * _TK - K
    keys_p = jnp.pad((keys * 2.0).astype(jnp.bfloat16), ((0, pad), (0, 0)))
    ksq_p = jnp.pad(ksq, ((0, pad), (0, 0)), constant_values=_BIGF)
    vals_t, idx_t = pl.pallas_call(
        functools.partial(_retr_kernel, nk),
        grid=(nq, nk + 1),
        in_specs=[
            pl.BlockSpec((_TQ, Din), lambda qi, ki: (qi, 0)),
            pl.BlockSpec((_TK, D), lambda qi, ki: (jnp.minimum(ki, nk - 1), 0)),
            pl.BlockSpec((Din, D), lambda qi, ki: (0, 0)),
            pl.BlockSpec((1, D), lambda qi, ki: (0, 0)),
            pl.BlockSpec((_TK, 1), lambda qi, ki: (jnp.maximum(ki - 1, 0), 0)),
        ],
        out_specs=[
            pl.BlockSpec((1, _TOPK, _TQ), lambda qi, ki: (qi, 0, 0)),
            pl.BlockSpec((1, _TOPK, _TQ), lambda qi, ki: (qi, 0, 0)),
        ],
        out_shape=[
            jax.ShapeDtypeStruct((nq, _TOPK, _TQ), jnp.float32),
            jax.ShapeDtypeStruct((nq, _TOPK, _TQ), jnp.int32),
        ],
        scratch_shapes=[
            pltpu.VMEM((_TQ, D), jnp.bfloat16),
            pltpu.VMEM((1, _TQ), jnp.float32),
            pltpu.VMEM((_TOPK, _TQ), jnp.float32),
            pltpu.VMEM((_TOPK, _TQ), jnp.int32),
            pltpu.VMEM((2, _TK, _TQ), jnp.float32),
        ],
    )(image_emb, keys_p, wt, b2, ksq_p)
    vals = vals_t.transpose(0, 2, 1).reshape(Q, _TOPK)
    idx = idx_t.transpose(0, 2, 1).reshape(Q, _TOPK)
    return vals, idx


# speedup vs baseline: 1.0983x; 1.0983x over previous
import jax
import jax.numpy as jnp
from jax.experimental import pallas as pl
from jax.experimental.pallas import tpu as pltpu

_TOPK = 3
_BIGF = 3.0e38
_TQ = 256
_TK = 4096


def _lexmin_tree(v, i):
    while v.shape[0] >= 16:
        q = v.shape[0] // 4
        v0, v1, v2, v3 = v[:q], v[q:2 * q], v[2 * q:3 * q], v[3 * q:]
        j0, j1, j2, j3 = i[:q], i[q:2 * q], i[2 * q:3 * q], i[3 * q:]
        ca = v1 < v0
        va = jnp.where(ca, v1, v0)
        ja = jnp.where(ca, j1, j0)
        cb = v3 < v2
        vb = jnp.where(cb, v3, v2)
        jb = jnp.where(cb, j3, j2)
        cc = vb < va
        v = jnp.where(cc, vb, va)
        i = jnp.where(cc, jb, ja)
    while v.shape[0] > 1:
        h = v.shape[0] // 2
        c = v[h:] < v[:h]
        i = jnp.where(c, i[h:], i[:h])
        v = jnp.where(c, v[h:], v[:h])
    return v, i


def _tile_top3(d, ic):
    n = d.shape[0]
    q = n // 4
    ia = ic[:q]
    vals, idxs = [], []
    for j in range(_TOPK):
        v0, v1, v2, v3 = d[:q], d[q:2 * q], d[2 * q:3 * q], d[3 * q:]
        ca = v1 < v0
        va = jnp.where(ca, v1, v0)
        ja = jnp.where(ca, ia + q, ia)
        cb = v3 < v2
        vb = jnp.where(cb, v3, v2)
        jb = jnp.where(cb, ia + 3 * q, ia + 2 * q)
        cc = vb < va
        v, i = _lexmin_tree(jnp.where(cc, vb, va), jnp.where(cc, jb, ja))
        vals.append(v)
        idxs.append(i)
        if j < _TOPK - 1:
            d = jnp.where(ic == i, _BIGF, d)
    return jnp.concatenate(vals, axis=0), jnp.concatenate(idxs, axis=0)


def _merge_top3(cv, ci):
    vals, idxs = [], []
    pos = jax.lax.broadcasted_iota(jnp.int32, (8, 1), 0)
    for j in range(_TOPK):
        v, i = _lexmin_tree(cv, ci)
        _, p = _lexmin_tree(cv, jnp.broadcast_to(pos, cv.shape))
        vals.append(v)
        idxs.append(i)
        if j < _TOPK - 1:
            cv = jnp.where(pos == p, _BIGF, cv)
    return jnp.concatenate(vals, axis=0), jnp.concatenate(idxs, axis=0)


def _retr_kernel(img_ref, keys_ref, wt_ref, b_ref, ksq_ref,
                 outv_ref, outi_ref, proj_ref, qsq_ref, rv_ref, ri_ref):
    ki = pl.program_id(0)
    qi = pl.program_id(1)
    sl = pl.ds(qi * _TQ, _TQ)

    @pl.when(ki == 0)
    def _project():
        p = jax.lax.dot_general(
            img_ref[...], wt_ref[...], (((1,), (0,)), ((), ())),
            preferred_element_type=jnp.float32) + b_ref[...]
        proj_ref[sl, :] = p.astype(jnp.bfloat16)
        qsq_ref[qi] = jax.lax.dot_general(
            jnp.ones((1, p.shape[1]), jnp.float32), p * p,
            (((1,), (1,)), ((), ())),
            preferred_element_type=jnp.float32,
            precision=jax.lax.Precision.HIGHEST)

    p = proj_ref[sl, :]
    kb2 = keys_ref[...]
    mm2 = jax.lax.dot_general(kb2, p, (((1,), (1,)), ((), ())),
                              preferred_element_type=jnp.float32)
    d = (qsq_ref[qi] + ksq_ref[...]) - mm2

    ic = jax.lax.broadcasted_iota(jnp.int32, (_TK, 1), 0)
    tv, ti = _tile_top3(d, ic)
    ti = ki * _TK + ti

    @pl.when(ki == 0)
    def _init():
        rv_ref[qi] = tv
        ri_ref[qi] = ti

    @pl.when(ki != 0)
    def _merge():
        padv = jnp.full((2, _TQ), _BIGF, jnp.float32)
        padi = jnp.zeros((2, _TQ), jnp.int32)
        cv = jnp.concatenate([rv_ref[qi], tv, padv], axis=0)
        ci = jnp.concatenate([ri_ref[qi], ti, padi], axis=0)
        nv, ni = _merge_top3(cv, ci)
        rv_ref[qi] = nv
        ri_ref[qi] = ni

    outv_ref[qi] = -rv_ref[qi]
    outi_ref[qi] = ri_ref[qi]


def kernel(image_emb, keys, W, b):
    Q, Din = image_emb.shape
    K, D = keys.shape
    nq = Q // _TQ
    nk = (K + _TK - 1) // _TK
    wt = W.T
    b2 = b.reshape(1, D)
    ksq = jnp.sum(keys * keys, axis=1)[:, None]
    pad = nk * _TK - K
    keys_p = jnp.pad((keys * 2.0).astype(jnp.bfloat16), ((0, pad), (0, 0)))
    ksq_p = jnp.pad(ksq, ((0, pad), (0, 0)), constant_values=_BIGF)
    vals_t, idx_t = pl.pallas_call(
        _retr_kernel,
        grid=(nk, nq),
        in_specs=[
            pl.BlockSpec((_TQ, Din), lambda ki, qi: (qi, 0)),
            pl.BlockSpec((_TK, D), lambda ki, qi: (ki, 0)),
            pl.BlockSpec((Din, D), lambda ki, qi: (0, 0)),
            pl.BlockSpec((1, D), lambda ki, qi: (0, 0)),
            pl.BlockSpec((_TK, 1), lambda ki, qi: (ki, 0)),
        ],
        out_specs=[
            pl.BlockSpec((nq, _TOPK, _TQ), lambda ki, qi: (0, 0, 0)),
            pl.BlockSpec((nq, _TOPK, _TQ), lambda ki, qi: (0, 0, 0)),
        ],
        out_shape=[
            jax.ShapeDtypeStruct((nq, _TOPK, _TQ), jnp.float32),
            jax.ShapeDtypeStruct((nq, _TOPK, _TQ), jnp.int32),
        ],
        scratch_shapes=[
            pltpu.VMEM((Q, D), jnp.bfloat16),
            pltpu.VMEM((nq, 1, _TQ), jnp.float32),
            pltpu.VMEM((nq, _TOPK, _TQ), jnp.float32),
            pltpu.VMEM((nq, _TOPK, _TQ), jnp.int32),
        ],
    )(image_emb, keys_p, wt, b2, ksq_p)
    vals = vals_t.transpose(0, 2, 1).reshape(Q, _TOPK)
    idx = idx_t.transpose(0, 2, 1).reshape(Q, _TOPK)
    return vals, idx


# speedup vs baseline: 1.1027x; 1.0040x over previous
import jax
import jax.numpy as jnp
from jax.experimental import pallas as pl
from jax.experimental.pallas import tpu as pltpu

_TOPK = 3
_BIGF = 3.0e38
_TQ = 256
_TK = 2048


def _lexmin_tree(v, i):
    while v.shape[0] >= 16:
        q = v.shape[0] // 4
        v0, v1, v2, v3 = v[:q], v[q:2 * q], v[2 * q:3 * q], v[3 * q:]
        j0, j1, j2, j3 = i[:q], i[q:2 * q], i[2 * q:3 * q], i[3 * q:]
        ca = v1 < v0
        va = jnp.where(ca, v1, v0)
        ja = jnp.where(ca, j1, j0)
        cb = v3 < v2
        vb = jnp.where(cb, v3, v2)
        jb = jnp.where(cb, j3, j2)
        cc = vb < va
        v = jnp.where(cc, vb, va)
        i = jnp.where(cc, jb, ja)
    while v.shape[0] > 1:
        h = v.shape[0] // 2
        c = v[h:] < v[:h]
        i = jnp.where(c, i[h:], i[:h])
        v = jnp.where(c, v[h:], v[:h])
    return v, i


def _tile_top3(d, ic):
    n = d.shape[0]
    q = n // 4
    ia = ic[:q]
    vals, idxs = [], []
    for j in range(_TOPK):
        v0, v1, v2, v3 = d[:q], d[q:2 * q], d[2 * q:3 * q], d[3 * q:]
        ca = v1 < v0
        va = jnp.where(ca, v1, v0)
        ja = jnp.where(ca, ia + q, ia)
        cb = v3 < v2
        vb = jnp.where(cb, v3, v2)
        jb = jnp.where(cb, ia + 3 * q, ia + 2 * q)
        cc = vb < va
        v, i = _lexmin_tree(jnp.where(cc, vb, va), jnp.where(cc, jb, ja))
        vals.append(v)
        idxs.append(i)
        if j < _TOPK - 1:
            d = jnp.where(ic == i, _BIGF, d)
    return jnp.concatenate(vals, axis=0), jnp.concatenate(idxs, axis=0)


def _merge_top3(cv, ci):
    vals, idxs = [], []
    pos = jax.lax.broadcasted_iota(jnp.int32, (8, 1), 0)
    for j in range(_TOPK):
        v, i = _lexmin_tree(cv, ci)
        _, p = _lexmin_tree(cv, jnp.broadcast_to(pos, cv.shape))
        vals.append(v)
        idxs.append(i)
        if j < _TOPK - 1:
            cv = jnp.where(pos == p, _BIGF, cv)
    return jnp.concatenate(vals, axis=0), jnp.concatenate(idxs, axis=0)


def _retr_kernel(img_ref, keys_ref, wt_ref, b_ref, ksq_ref,
                 outv_ref, outi_ref, proj_ref, qsq_ref, rv_ref, ri_ref):
    ki = pl.program_id(0)
    qi = pl.program_id(1)
    sl = pl.ds(qi * _TQ, _TQ)

    @pl.when(ki == 0)
    def _project():
        p = jax.lax.dot_general(
            img_ref[...], wt_ref[...], (((1,), (0,)), ((), ())),
            preferred_element_type=jnp.float32) + b_ref[...]
        proj_ref[sl, :] = p.astype(jnp.bfloat16)
        qsq_ref[qi] = jax.lax.dot_general(
            jnp.ones((1, p.shape[1]), jnp.float32), p * p,
            (((1,), (1,)), ((), ())),
            preferred_element_type=jnp.float32,
            precision=jax.lax.Precision.HIGHEST)

    p = proj_ref[sl, :]
    kb2 = keys_ref[...]
    mm2 = jax.lax.dot_general(kb2, p, (((1,), (1,)), ((), ())),
                              preferred_element_type=jnp.float32)
    d = (qsq_ref[qi] + ksq_ref[...]) - mm2

    ic = jax.lax.broadcasted_iota(jnp.int32, (_TK, 1), 0)
    tv, ti = _tile_top3(d, ic)
    ti = ki * _TK + ti

    @pl.when(ki == 0)
    def _init():
        rv_ref[qi] = tv
        ri_ref[qi] = ti

    @pl.when(ki != 0)
    def _merge():
        padv = jnp.full((2, _TQ), _BIGF, jnp.float32)
        padi = jnp.zeros((2, _TQ), jnp.int32)
        cv = jnp.concatenate([rv_ref[qi], tv, padv], axis=0)
        ci = jnp.concatenate([ri_ref[qi], ti, padi], axis=0)
        nv, ni = _merge_top3(cv, ci)
        rv_ref[qi] = nv
        ri_ref[qi] = ni

    outv_ref[qi] = -rv_ref[qi]
    outi_ref[qi] = ri_ref[qi]


def kernel(image_emb, keys, W, b):
    Q, Din = image_emb.shape
    K, D = keys.shape
    nq = Q // _TQ
    nk = (K + _TK - 1) // _TK
    wt = W.T
    b2 = b.reshape(1, D)
    ksq = jnp.sum(keys * keys, axis=1)[:, None]
    pad = nk * _TK - K
    keys_p = jnp.pad((keys * 2.0).astype(jnp.bfloat16), ((0, pad), (0, 0)))
    ksq_p = jnp.pad(ksq, ((0, pad), (0, 0)), constant_values=_BIGF)
    vals_t, idx_t = pl.pallas_call(
        _retr_kernel,
        grid=(nk, nq),
        in_specs=[
            pl.BlockSpec((_TQ, Din), lambda ki, qi: (qi, 0)),
            pl.BlockSpec((_TK, D), lambda ki, qi: (ki, 0)),
            pl.BlockSpec((Din, D), lambda ki, qi: (0, 0)),
            pl.BlockSpec((1, D), lambda ki, qi: (0, 0)),
            pl.BlockSpec((_TK, 1), lambda ki, qi: (ki, 0)),
        ],
        out_specs=[
            pl.BlockSpec((nq, _TOPK, _TQ), lambda ki, qi: (0, 0, 0)),
            pl.BlockSpec((nq, _TOPK, _TQ), lambda ki, qi: (0, 0, 0)),
        ],
        out_shape=[
            jax.ShapeDtypeStruct((nq, _TOPK, _TQ), jnp.float32),
            jax.ShapeDtypeStruct((nq, _TOPK, _TQ), jnp.int32),
        ],
        scratch_shapes=[
            pltpu.VMEM((Q, D), jnp.bfloat16),
            pltpu.VMEM((nq, 1, _TQ), jnp.float32),
            pltpu.VMEM((nq, _TOPK, _TQ), jnp.float32),
            pltpu.VMEM((nq, _TOPK, _TQ), jnp.int32),
        ],
    )(image_emb, keys_p, wt, b2, ksq_p)
    vals = vals_t.transpose(0, 2, 1).reshape(Q, _TOPK)
    idx = idx_t.transpose(0, 2, 1).reshape(Q, _TOPK)
    return vals, idx


# speedup vs baseline: 1.1613x; 1.0532x over previous
import jax
import jax.numpy as jnp
from jax.experimental import pallas as pl
from jax.experimental.pallas import tpu as pltpu

_TOPK = 3
_BIGF = 3.0e38
_TQ = 512
_TK = 2048


def _lexmin_tree(v, i):
    while v.shape[0] >= 16:
        q = v.shape[0] // 4
        v0, v1, v2, v3 = v[:q], v[q:2 * q], v[2 * q:3 * q], v[3 * q:]
        j0, j1, j2, j3 = i[:q], i[q:2 * q], i[2 * q:3 * q], i[3 * q:]
        ca = v1 < v0
        va = jnp.where(ca, v1, v0)
        ja = jnp.where(ca, j1, j0)
        cb = v3 < v2
        vb = jnp.where(cb, v3, v2)
        jb = jnp.where(cb, j3, j2)
        cc = vb < va
        v = jnp.where(cc, vb, va)
        i = jnp.where(cc, jb, ja)
    while v.shape[0] > 1:
        h = v.shape[0] // 2
        c = v[h:] < v[:h]
        i = jnp.where(c, i[h:], i[:h])
        v = jnp.where(c, v[h:], v[:h])
    return v, i


def _tile_top3(d, ic):
    n = d.shape[0]
    q = n // 4
    ia = ic[:q]
    vals, idxs = [], []
    for j in range(_TOPK):
        v0, v1, v2, v3 = d[:q], d[q:2 * q], d[2 * q:3 * q], d[3 * q:]
        ca = v1 < v0
        va = jnp.where(ca, v1, v0)
        ja = jnp.where(ca, ia + q, ia)
        cb = v3 < v2
        vb = jnp.where(cb, v3, v2)
        jb = jnp.where(cb, ia + 3 * q, ia + 2 * q)
        cc = vb < va
        v, i = _lexmin_tree(jnp.where(cc, vb, va), jnp.where(cc, jb, ja))
        vals.append(v)
        idxs.append(i)
        if j < _TOPK - 1:
            d = jnp.where(ic == i, _BIGF, d)
    return jnp.concatenate(vals, axis=0), jnp.concatenate(idxs, axis=0)


def _merge_top3(cv, ci):
    vals, idxs = [], []
    pos = jax.lax.broadcasted_iota(jnp.int32, (8, 1), 0)
    for j in range(_TOPK):
        v, i = _lexmin_tree(cv, ci)
        _, p = _lexmin_tree(cv, jnp.broadcast_to(pos, cv.shape))
        vals.append(v)
        idxs.append(i)
        if j < _TOPK - 1:
            cv = jnp.where(pos == p, _BIGF, cv)
    return jnp.concatenate(vals, axis=0), jnp.concatenate(idxs, axis=0)


def _retr_kernel(img_ref, keys_ref, wt_ref, b_ref, ksq_ref,
                 outv_ref, outi_ref, proj_ref, qsq_ref, rv_ref, ri_ref):
    ki = pl.program_id(0)
    qi = pl.program_id(1)
    sl = pl.ds(qi * _TQ, _TQ)

    @pl.when(ki == 0)
    def _project():
        p = jax.lax.dot_general(
            img_ref[...], wt_ref[...], (((1,), (0,)), ((), ())),
            preferred_element_type=jnp.float32) + b_ref[...]
        proj_ref[sl, :] = p.astype(jnp.bfloat16)
        qsq_ref[qi] = jax.lax.dot_general(
            jnp.ones((1, p.shape[1]), jnp.float32), p * p,
            (((1,), (1,)), ((), ())),
            preferred_element_type=jnp.float32,
            precision=jax.lax.Precision.HIGHEST)

    p = proj_ref[sl, :]
    kb2 = keys_ref[...]
    mm2 = jax.lax.dot_general(kb2, p, (((1,), (1,)), ((), ())),
                              preferred_element_type=jnp.float32)
    d = (qsq_ref[qi] + ksq_ref[...]) - mm2

    ic = jax.lax.broadcasted_iota(jnp.int32, (_TK, 1), 0)
    tv, ti = _tile_top3(d, ic)
    ti = ki * _TK + ti

    @pl.when(ki == 0)
    def _init():
        rv_ref[qi] = tv
        ri_ref[qi] = ti

    @pl.when(ki != 0)
    def _merge():
        padv = jnp.full((2, _TQ), _BIGF, jnp.float32)
        padi = jnp.zeros((2, _TQ), jnp.int32)
        cv = jnp.concatenate([rv_ref[qi], tv, padv], axis=0)
        ci = jnp.concatenate([ri_ref[qi], ti, padi], axis=0)
        nv, ni = _merge_top3(cv, ci)
        rv_ref[qi] = nv
        ri_ref[qi] = ni

    outv_ref[qi] = -rv_ref[qi]
    outi_ref[qi] = ri_ref[qi]


def kernel(image_emb, keys, W, b):
    Q, Din = image_emb.shape
    K, D = keys.shape
    nq = Q // _TQ
    nk = (K + _TK - 1) // _TK
    wt = W.T
    b2 = b.reshape(1, D)
    ksq = jnp.sum(keys * keys, axis=1)[:, None]
    pad = nk * _TK - K
    keys_p = jnp.pad((keys * 2.0).astype(jnp.bfloat16), ((0, pad), (0, 0)))
    ksq_p = jnp.pad(ksq, ((0, pad), (0, 0)), constant_values=_BIGF)
    vals_t, idx_t = pl.pallas_call(
        _retr_kernel,
        grid=(nk, nq),
        in_specs=[
            pl.BlockSpec((_TQ, Din), lambda ki, qi: (qi, 0)),
            pl.BlockSpec((_TK, D), lambda ki, qi: (ki, 0)),
            pl.BlockSpec((Din, D), lambda ki, qi: (0, 0)),
            pl.BlockSpec((1, D), lambda ki, qi: (0, 0)),
            pl.BlockSpec((_TK, 1), lambda ki, qi: (ki, 0)),
        ],
        out_specs=[
            pl.BlockSpec((nq, _TOPK, _TQ), lambda ki, qi: (0, 0, 0)),
            pl.BlockSpec((nq, _TOPK, _TQ), lambda ki, qi: (0, 0, 0)),
        ],
        out_shape=[
            jax.ShapeDtypeStruct((nq, _TOPK, _TQ), jnp.float32),
            jax.ShapeDtypeStruct((nq, _TOPK, _TQ), jnp.int32),
        ],
        scratch_shapes=[
            pltpu.VMEM((Q, D), jnp.bfloat16),
            pltpu.VMEM((nq, 1, _TQ), jnp.float32),
            pltpu.VMEM((nq, _TOPK, _TQ), jnp.float32),
            pltpu.VMEM((nq, _TOPK, _TQ), jnp.int32),
        ],
    )(image_emb, keys_p, wt, b2, ksq_p)
    vals = vals_t.transpose(0, 2, 1).reshape(Q, _TOPK)
    idx = idx_t.transpose(0, 2, 1).reshape(Q, _TOPK)
    return vals, idx


# speedup vs baseline: 1.2015x; 1.0346x over previous
import jax
import jax.numpy as jnp
from jax.experimental import pallas as pl
from jax.experimental.pallas import tpu as pltpu

_TOPK = 3
_BIGF = 3.0e38
_TQ = 1024
_TK = 2048


def _lexmin_tree(v, i):
    while v.shape[0] >= 16:
        q = v.shape[0] // 4
        v0, v1, v2, v3 = v[:q], v[q:2 * q], v[2 * q:3 * q], v[3 * q:]
        j0, j1, j2, j3 = i[:q], i[q:2 * q], i[2 * q:3 * q], i[3 * q:]
        ca = v1 < v0
        va = jnp.where(ca, v1, v0)
        ja = jnp.where(ca, j1, j0)
        cb = v3 < v2
        vb = jnp.where(cb, v3, v2)
        jb = jnp.where(cb, j3, j2)
        cc = vb < va
        v = jnp.where(cc, vb, va)
        i = jnp.where(cc, jb, ja)
    while v.shape[0] > 1:
        h = v.shape[0] // 2
        c = v[h:] < v[:h]
        i = jnp.where(c, i[h:], i[:h])
        v = jnp.where(c, v[h:], v[:h])
    return v, i


def _tile_top3(d, ic):
    n = d.shape[0]
    q = n // 4
    ia = ic[:q]
    vals, idxs = [], []
    for j in range(_TOPK):
        v0, v1, v2, v3 = d[:q], d[q:2 * q], d[2 * q:3 * q], d[3 * q:]
        ca = v1 < v0
        va = jnp.where(ca, v1, v0)
        ja = jnp.where(ca, ia + q, ia)
        cb = v3 < v2
        vb = jnp.where(cb, v3, v2)
        jb = jnp.where(cb, ia + 3 * q, ia + 2 * q)
        cc = vb < va
        v, i = _lexmin_tree(jnp.where(cc, vb, va), jnp.where(cc, jb, ja))
        vals.append(v)
        idxs.append(i)
        if j < _TOPK - 1:
            d = jnp.where(ic == i, _BIGF, d)
    return jnp.concatenate(vals, axis=0), jnp.concatenate(idxs, axis=0)


def _merge_top3(cv, ci):
    vals, idxs = [], []
    pos = jax.lax.broadcasted_iota(jnp.int32, (8, 1), 0)
    for j in range(_TOPK):
        v, i = _lexmin_tree(cv, ci)
        _, p = _lexmin_tree(cv, jnp.broadcast_to(pos, cv.shape))
        vals.append(v)
        idxs.append(i)
        if j < _TOPK - 1:
            cv = jnp.where(pos == p, _BIGF, cv)
    return jnp.concatenate(vals, axis=0), jnp.concatenate(idxs, axis=0)


def _retr_kernel(img_ref, keys_ref, wt_ref, b_ref, ksq_ref,
                 outv_ref, outi_ref, proj_ref, qsq_ref, rv_ref, ri_ref):
    ki = pl.program_id(0)
    qi = pl.program_id(1)
    sl = pl.ds(qi * _TQ, _TQ)

    @pl.when(ki == 0)
    def _project():
        p = jax.lax.dot_general(
            img_ref[...], wt_ref[...], (((1,), (0,)), ((), ())),
            preferred_element_type=jnp.float32) + b_ref[...]
        proj_ref[sl, :] = p.astype(jnp.bfloat16)
        qsq_ref[qi] = jax.lax.dot_general(
            jnp.ones((1, p.shape[1]), jnp.float32), p * p,
            (((1,), (1,)), ((), ())),
            preferred_element_type=jnp.float32,
            precision=jax.lax.Precision.HIGHEST)

    p = proj_ref[sl, :]
    kb2 = keys_ref[...]
    mm2 = jax.lax.dot_general(kb2, p, (((1,), (1,)), ((), ())),
                              preferred_element_type=jnp.float32)
    d = (qsq_ref[qi] + ksq_ref[...]) - mm2

    ic = jax.lax.broadcasted_iota(jnp.int32, (_TK, 1), 0)
    tv, ti = _tile_top3(d, ic)
    ti = ki * _TK + ti

    @pl.when(ki == 0)
    def _init():
        rv_ref[qi] = tv
        ri_ref[qi] = ti

    @pl.when(ki != 0)
    def _merge():
        padv = jnp.full((2, _TQ), _BIGF, jnp.float32)
        padi = jnp.zeros((2, _TQ), jnp.int32)
        cv = jnp.concatenate([rv_ref[qi], tv, padv], axis=0)
        ci = jnp.concatenate([ri_ref[qi], ti, padi], axis=0)
        nv, ni = _merge_top3(cv, ci)
        rv_ref[qi] = nv
        ri_ref[qi] = ni

    outv_ref[qi] = -rv_ref[qi]
    outi_ref[qi] = ri_ref[qi]


def kernel(image_emb, keys, W, b):
    Q, Din = image_emb.shape
    K, D = keys.shape
    nq = Q // _TQ
    nk = (K + _TK - 1) // _TK
    wt = W.T
    b2 = b.reshape(1, D)
    ksq = jnp.sum(keys * keys, axis=1)[:, None]
    pad = nk * _TK - K
    keys_p = jnp.pad((keys * 2.0).astype(jnp.bfloat16), ((0, pad), (0, 0)))
    ksq_p = jnp.pad(ksq, ((0, pad), (0, 0)), constant_values=_BIGF)
    vals_t, idx_t = pl.pallas_call(
        _retr_kernel,
        grid=(nk, nq),
        in_specs=[
            pl.BlockSpec((_TQ, Din), lambda ki, qi: (qi, 0)),
            pl.BlockSpec((_TK, D), lambda ki, qi: (ki, 0)),
            pl.BlockSpec((Din, D), lambda ki, qi: (0, 0)),
            pl.BlockSpec((1, D), lambda ki, qi: (0, 0)),
            pl.BlockSpec((_TK, 1), lambda ki, qi: (ki, 0)),
        ],
        out_specs=[
            pl.BlockSpec((nq, _TOPK, _TQ), lambda ki, qi: (0, 0, 0)),
            pl.BlockSpec((nq, _TOPK, _TQ), lambda ki, qi: (0, 0, 0)),
        ],
        out_shape=[
            jax.ShapeDtypeStruct((nq, _TOPK, _TQ), jnp.float32),
            jax.ShapeDtypeStruct((nq, _TOPK, _TQ), jnp.int32),
        ],
        scratch_shapes=[
            pltpu.VMEM((Q, D), jnp.bfloat16),
            pltpu.VMEM((nq, 1, _TQ), jnp.float32),
            pltpu.VMEM((nq, _TOPK, _TQ), jnp.float32),
            pltpu.VMEM((nq, _TOPK, _TQ), jnp.int32),
        ],
    )(image_emb, keys_p, wt, b2, ksq_p)
    vals = vals_t.transpose(0, 2, 1).reshape(Q, _TOPK)
    idx = idx_t.transpose(0, 2, 1).reshape(Q, _TOPK)
    return vals, idx


# speedup vs baseline: 1.2360x; 1.0287x over previous
import jax
import jax.numpy as jnp
from jax.experimental import pallas as pl
from jax.experimental.pallas import tpu as pltpu

_TOPK = 3
_BIGF = 3.0e38
_TQ = 2048
_TK = 2048


def _lexmin_tree(v, i):
    while v.shape[0] >= 16:
        q = v.shape[0] // 4
        v0, v1, v2, v3 = v[:q], v[q:2 * q], v[2 * q:3 * q], v[3 * q:]
        j0, j1, j2, j3 = i[:q], i[q:2 * q], i[2 * q:3 * q], i[3 * q:]
        ca = v1 < v0
        va = jnp.where(ca, v1, v0)
        ja = jnp.where(ca, j1, j0)
        cb = v3 < v2
        vb = jnp.where(cb, v3, v2)
        jb = jnp.where(cb, j3, j2)
        cc = vb < va
        v = jnp.where(cc, vb, va)
        i = jnp.where(cc, jb, ja)
    while v.shape[0] > 1:
        h = v.shape[0] // 2
        c = v[h:] < v[:h]
        i = jnp.where(c, i[h:], i[:h])
        v = jnp.where(c, v[h:], v[:h])
    return v, i


def _tile_top3(d, ic):
    n = d.shape[0]
    q = n // 4
    ia = ic[:q]
    vals, idxs = [], []
    for j in range(_TOPK):
        v0, v1, v2, v3 = d[:q], d[q:2 * q], d[2 * q:3 * q], d[3 * q:]
        ca = v1 < v0
        va = jnp.where(ca, v1, v0)
        ja = jnp.where(ca, ia + q, ia)
        cb = v3 < v2
        vb = jnp.where(cb, v3, v2)
        jb = jnp.where(cb, ia + 3 * q, ia + 2 * q)
        cc = vb < va
        v, i = _lexmin_tree(jnp.where(cc, vb, va), jnp.where(cc, jb, ja))
        vals.append(v)
        idxs.append(i)
        if j < _TOPK - 1:
            d = jnp.where(ic == i, _BIGF, d)
    return jnp.concatenate(vals, axis=0), jnp.concatenate(idxs, axis=0)


def _merge_top3(cv, ci):
    vals, idxs = [], []
    pos = jax.lax.broadcasted_iota(jnp.int32, (8, 1), 0)
    for j in range(_TOPK):
        v, i = _lexmin_tree(cv, ci)
        _, p = _lexmin_tree(cv, jnp.broadcast_to(pos, cv.shape))
        vals.append(v)
        idxs.append(i)
        if j < _TOPK - 1:
            cv = jnp.where(pos == p, _BIGF, cv)
    return jnp.concatenate(vals, axis=0), jnp.concatenate(idxs, axis=0)


def _retr_kernel(img_ref, keys_ref, wt_ref, b_ref, ksq_ref,
                 outv_ref, outi_ref, proj_ref, qsq_ref, rv_ref, ri_ref):
    ki = pl.program_id(0)
    qi = pl.program_id(1)
    sl = pl.ds(qi * _TQ, _TQ)

    @pl.when(ki == 0)
    def _project():
        p = jax.lax.dot_general(
            img_ref[...], wt_ref[...], (((1,), (0,)), ((), ())),
            preferred_element_type=jnp.float32) + b_ref[...]
        proj_ref[sl, :] = p.astype(jnp.bfloat16)
        qsq_ref[qi] = jax.lax.dot_general(
            jnp.ones((1, p.shape[1]), jnp.float32), p * p,
            (((1,), (1,)), ((), ())),
            preferred_element_type=jnp.float32,
            precision=jax.lax.Precision.HIGHEST)

    p = proj_ref[sl, :]
    kb2 = keys_ref[...]
    mm2 = jax.lax.dot_general(kb2, p, (((1,), (1,)), ((), ())),
                              preferred_element_type=jnp.float32)
    d = (qsq_ref[qi] + ksq_ref[...]) - mm2

    ic = jax.lax.broadcasted_iota(jnp.int32, (_TK, 1), 0)
    tv, ti = _tile_top3(d, ic)
    ti = ki * _TK + ti

    @pl.when(ki == 0)
    def _init():
        rv_ref[qi] = tv
        ri_ref[qi] = ti

    @pl.when(ki != 0)
    def _merge():
        padv = jnp.full((2, _TQ), _BIGF, jnp.float32)
        padi = jnp.zeros((2, _TQ), jnp.int32)
        cv = jnp.concatenate([rv_ref[qi], tv, padv], axis=0)
        ci = jnp.concatenate([ri_ref[qi], ti, padi], axis=0)
        nv, ni = _merge_top3(cv, ci)
        rv_ref[qi] = nv
        ri_ref[qi] = ni

    outv_ref[qi] = -rv_ref[qi]
    outi_ref[qi] = ri_ref[qi]


def kernel(image_emb, keys, W, b):
    Q, Din = image_emb.shape
    K, D = keys.shape
    nq = Q // _TQ
    nk = (K + _TK - 1) // _TK
    wt = W.T
    b2 = b.reshape(1, D)
    ksq = jnp.sum(keys * keys, axis=1)[:, None]
    pad = nk * _TK - K
    keys_p = jnp.pad((keys * 2.0).astype(jnp.bfloat16), ((0, pad), (0, 0)))
    ksq_p = jnp.pad(ksq, ((0, pad), (0, 0)), constant_values=_BIGF)
    vals_t, idx_t = pl.pallas_call(
        _retr_kernel,
        grid=(nk, nq),
        in_specs=[
            pl.BlockSpec((_TQ, Din), lambda ki, qi: (qi, 0)),
            pl.BlockSpec((_TK, D), lambda ki, qi: (ki, 0)),
            pl.BlockSpec((Din, D), lambda ki, qi: (0, 0)),
            pl.BlockSpec((1, D), lambda ki, qi: (0, 0)),
            pl.BlockSpec((_TK, 1), lambda ki, qi: (ki, 0)),
        ],
        out_specs=[
            pl.BlockSpec((nq, _TOPK, _TQ), lambda ki, qi: (0, 0, 0)),
            pl.BlockSpec((nq, _TOPK, _TQ), lambda ki, qi: (0, 0, 0)),
        ],
        out_shape=[
            jax.ShapeDtypeStruct((nq, _TOPK, _TQ), jnp.float32),
            jax.ShapeDtypeStruct((nq, _TOPK, _TQ), jnp.int32),
        ],
        scratch_shapes=[
            pltpu.VMEM((Q, D), jnp.bfloat16),
            pltpu.VMEM((nq, 1, _TQ), jnp.float32),
            pltpu.VMEM((nq, _TOPK, _TQ), jnp.float32),
            pltpu.VMEM((nq, _TOPK, _TQ), jnp.int32),
        ],
    )(image_emb, keys_p, wt, b2, ksq_p)
    vals = vals_t.transpose(0, 2, 1).reshape(Q, _TOPK)
    idx = idx_t.transpose(0, 2, 1).reshape(Q, _TOPK)
    return vals, idx
